# Initial kernel scaffold; baseline (speedup 1.0000x reference)
import functools

import jax
import jax.numpy as jnp
from jax import lax
from jax.experimental import pallas as pl
from jax.experimental.pallas import tpu as pltpu
from jax.experimental.pallas import tpu_sc as plsc

_N = 100000
_E = 1600000
_F = 128
_R = _E // 128
_NW = 32
_ROWS = 391
_NPAD = 100096
_CHUNK = _NPAD // 16


def _sc_segment_body(dst_hbm, he_hbm, out_hbm, idx_v, val_v, zbuf, acc):
    c = lax.axis_index("c")
    s = lax.axis_index("s")
    w = s * 2 + c

    base = (w * _R) // _NW
    cnt = ((w + 1) * _R) // _NW - base

    pltpu.sync_copy(dst_hbm.at[pl.ds(base, _ROWS)], idx_v.at[pl.ds(0, _ROWS)])
    pltpu.sync_copy(he_hbm.at[pl.ds(base, _ROWS)], val_v.at[pl.ds(0, _ROWS)])

    z16f = jnp.zeros((16,), jnp.float32)
    z16i = jnp.zeros((16,), jnp.int32)
    for i in range(8):
        idx_v[_ROWS, pl.ds(i * 16, 16)] = z16i
        val_v[_ROWS, pl.ds(i * 16, 16)] = z16f

    @pl.when(cnt == 390)
    def _():
        for i in range(8):
            val_v[_ROWS - 1, pl.ds(i * 16, 16)] = z16f

    def zero_body(i, carry):
        zbuf[pl.ds(i * 16, 16)] = z16f
        return carry

    lax.fori_loop(0, _CHUNK // 16, zero_body, 0)
    pltpu.sync_copy(zbuf, acc.at[pl.ds(s * _CHUNK, _CHUNK)])
    plsc.subcore_barrier()

    def scat_body(j, carry):
        pltpu.sync_copy(val_v.at[j], acc.at[idx_v.at[j]], add=True)
        return carry

    lax.fori_loop(0, _ROWS + 1, scat_body, 0)
    plsc.subcore_barrier()

    pltpu.sync_copy(acc.at[pl.ds(s * _CHUNK, _CHUNK)], zbuf)
    pltpu.sync_copy(zbuf, out_hbm.at[c].at[pl.ds(s * _CHUNK, _CHUNK)])


@functools.partial(
    pl.kernel,
    out_type=jax.ShapeDtypeStruct((2, _NPAD), jnp.float32),
    mesh=plsc.VectorSubcoreMesh(core_axis_name="c", subcore_axis_name="s"),
    scratch_types=[
        pltpu.VMEM((_ROWS + 1, 128), jnp.int32),
        pltpu.VMEM((_ROWS + 1, 128), jnp.float32),
        pltpu.VMEM((_CHUNK,), jnp.float32),
        pltpu.VMEM_SHARED((_NPAD,), jnp.float32),
    ],
)
def _sc_segment(dst_hbm, he_hbm, out_hbm, idx_v, val_v, zbuf, acc):
    _sc_segment_body(dst_hbm, he_hbm, out_hbm, idx_v, val_v, zbuf, acc)


_BN = 2048


def _mlp_body(pt_ref, hn_ref, w1a_ref, w1b_ref, b1_ref, w2_ref, b2_ref,
              w3_ref, b3_ref, out_ref):
    x = hn_ref[...]
    h = jnp.dot(x, w1a_ref[...], preferred_element_type=jnp.float32)
    h = h + pt_ref[...] * w1b_ref[...]
    h = jnp.maximum(h + b1_ref[...], 0.0)
    h = jnp.dot(h, w2_ref[...], preferred_element_type=jnp.float32)
    h = jnp.maximum(h + b2_ref[...], 0.0)
    out_ref[...] = (
        jnp.dot(h, w3_ref[...], preferred_element_type=jnp.float32)
        + b3_ref[...]
    )


def _mlp_tc(pt, hn, w1a, w1b, b1, w2, b2, w3, b3):
    n_out = w3.shape[1]
    grid = (pl.cdiv(_N, _BN),)
    return pl.pallas_call(
        _mlp_body,
        grid=grid,
        in_specs=[
            pl.BlockSpec((_BN, 1), lambda i: (i, 0)),
            pl.BlockSpec((_BN, _F), lambda i: (i, 0)),
            pl.BlockSpec((_F, _F), lambda i: (0, 0)),
            pl.BlockSpec((1, _F), lambda i: (0, 0)),
            pl.BlockSpec((1, _F), lambda i: (0, 0)),
            pl.BlockSpec((_F, _F), lambda i: (0, 0)),
            pl.BlockSpec((1, _F), lambda i: (0, 0)),
            pl.BlockSpec((_F, n_out), lambda i: (0, 0)),
            pl.BlockSpec((1, n_out), lambda i: (0, 0)),
        ],
        out_specs=pl.BlockSpec((_BN, n_out), lambda i: (i, 0)),
        out_shape=jax.ShapeDtypeStruct((_N, n_out), jnp.float32),
    )(pt, hn, w1a, w1b, b1, w2, b2, w3, b3)


def kernel(hn, he, edge_index, W1, b1, W2, b2, W3, b3):
    dst = edge_index[1].astype(jnp.int32).reshape(_R, 128)
    he2 = he.reshape(_R, 128)

    partials = _sc_segment(dst, he2)
    pt = (partials[0, :_N] + partials[1, :_N]).reshape(_N, 1)

    w1a = W1[:_F]
    w1b = W1[_F:_F + 1]
    return _mlp_tc(
        pt, hn, w1a, w1b, b1.reshape(1, _F), W2, b2.reshape(1, _F),
        W3, b3.reshape(1, -1),
    )


# speedup vs baseline: 6.4658x; 6.4658x over previous
import functools

import jax
import jax.numpy as jnp
from jax import lax
from jax.experimental import pallas as pl
from jax.experimental.pallas import tpu as pltpu
from jax.experimental.pallas import tpu_sc as plsc

_N = 100000
_E = 1600000
_F = 128
_R = _E // 128
_NW = 32
_ROWS = 391
_NPAD = 100096
_CHUNK = _NPAD // 16


def _sc_segment_body(dst_hbm, he_hbm, out_hbm, idx_v, val_v, zbuf, acc):
    c = lax.axis_index("c")
    s = lax.axis_index("s")
    w = s * 2 + c

    base = (w * _R) // _NW
    cnt = ((w + 1) * _R) // _NW - base

    pltpu.sync_copy(dst_hbm.at[pl.ds(base, _ROWS)], idx_v.at[pl.ds(0, _ROWS)])
    pltpu.sync_copy(he_hbm.at[pl.ds(base, _ROWS)], val_v.at[pl.ds(0, _ROWS)])

    z16f = jnp.zeros((16,), jnp.float32)
    z16i = jnp.zeros((16,), jnp.int32)
    for i in range(8):
        idx_v[_ROWS, pl.ds(i * 16, 16)] = z16i
        val_v[_ROWS, pl.ds(i * 16, 16)] = z16f

    @pl.when(cnt == 390)
    def _():
        for i in range(8):
            val_v[_ROWS - 1, pl.ds(i * 16, 16)] = z16f

    def zero_body(i, carry):
        zbuf[pl.ds(i * 16, 16)] = z16f
        return carry

    lax.fori_loop(0, _CHUNK // 16, zero_body, 0)
    pltpu.sync_copy(zbuf, acc.at[pl.ds(s * _CHUNK, _CHUNK)])
    plsc.subcore_barrier()

    def scat_body(j, carry):
        pltpu.sync_copy(val_v.at[j], acc.at[idx_v.at[j]], add=True)
        return carry

    lax.fori_loop(0, _ROWS + 1, scat_body, 0)
    plsc.subcore_barrier()

    pltpu.sync_copy(acc.at[pl.ds(s * _CHUNK, _CHUNK)], zbuf)
    pltpu.sync_copy(zbuf, out_hbm.at[c].at[pl.ds(s * _CHUNK, _CHUNK)])


@functools.cache
def _make_sc_segment():
    return pl.kernel(
        _sc_segment_body,
        out_type=jax.ShapeDtypeStruct((2, _NPAD), jnp.float32),
        mesh=plsc.VectorSubcoreMesh(core_axis_name="c", subcore_axis_name="s"),
        compiler_params=pltpu.CompilerParams(use_tc_tiling_on_sc=False),
        scratch_types=[
            pltpu.VMEM((_ROWS + 1, 128), jnp.int32),
            pltpu.VMEM((_ROWS + 1, 128), jnp.float32),
            pltpu.VMEM((_CHUNK,), jnp.float32),
            pltpu.VMEM_SHARED((_NPAD,), jnp.float32),
        ],
    )


_BN = 2048


def _mlp_body(pt_ref, hn_ref, w1a_ref, w1b_ref, b1_ref, w2_ref, b2_ref,
              w3_ref, b3_ref, out_ref):
    x = hn_ref[...]
    h = jnp.dot(x, w1a_ref[...], preferred_element_type=jnp.float32)
    h = h + pt_ref[...] * w1b_ref[...]
    h = jnp.maximum(h + b1_ref[...], 0.0)
    h = jnp.dot(h, w2_ref[...], preferred_element_type=jnp.float32)
    h = jnp.maximum(h + b2_ref[...], 0.0)
    out_ref[...] = (
        jnp.dot(h, w3_ref[...], preferred_element_type=jnp.float32)
        + b3_ref[...]
    )


def _mlp_tc(pt, hn, w1a, w1b, b1, w2, b2, w3, b3):
    n_out = w3.shape[1]
    grid = (pl.cdiv(_N, _BN),)
    return pl.pallas_call(
        _mlp_body,
        grid=grid,
        in_specs=[
            pl.BlockSpec((_BN, 1), lambda i: (i, 0)),
            pl.BlockSpec((_BN, _F), lambda i: (i, 0)),
            pl.BlockSpec((_F, _F), lambda i: (0, 0)),
            pl.BlockSpec((1, _F), lambda i: (0, 0)),
            pl.BlockSpec((1, _F), lambda i: (0, 0)),
            pl.BlockSpec((_F, _F), lambda i: (0, 0)),
            pl.BlockSpec((1, _F), lambda i: (0, 0)),
            pl.BlockSpec((_F, n_out), lambda i: (0, 0)),
            pl.BlockSpec((1, n_out), lambda i: (0, 0)),
        ],
        out_specs=pl.BlockSpec((_BN, n_out), lambda i: (i, 0)),
        out_shape=jax.ShapeDtypeStruct((_N, n_out), jnp.float32),
    )(pt, hn, w1a, w1b, b1, w2, b2, w3, b3)


def kernel(hn, he, edge_index, W1, b1, W2, b2, W3, b3):
    dst = edge_index[1].astype(jnp.int32).reshape(_R, 128)
    he2 = he.reshape(_R, 128)

    partials = _make_sc_segment()(dst, he2)
    pt = (partials[0, :_N] + partials[1, :_N]).reshape(_N, 1)

    w1a = W1[:_F]
    w1b = W1[_F:_F + 1]
    return _mlp_tc(
        pt, hn, w1a, w1b, b1.reshape(1, _F), W2, b2.reshape(1, _F),
        W3, b3.reshape(1, -1),
    )


# speedup vs baseline: 8.2683x; 1.2788x over previous
import functools

import jax
import jax.numpy as jnp
from jax import lax
from jax.experimental import pallas as pl
from jax.experimental.pallas import tpu as pltpu
from jax.experimental.pallas import tpu_sc as plsc

_N = 100000
_E = 1600000
_F = 128
_R = _E // 128
_NW = 32
_ROWS = 391
_NPAD = 100096
_CHUNK = _NPAD // 16


def _sc_segment_body(dst_hbm, he_hbm, out_hbm, idx_v, val_v, zbuf, acc):
    c = lax.axis_index("c")
    s = lax.axis_index("s")
    w = s * 2 + c

    base = (w * _R) // _NW
    cnt = ((w + 1) * _R) // _NW - base

    pltpu.sync_copy(dst_hbm.at[pl.ds(base, _ROWS)], idx_v.at[pl.ds(0, _ROWS)])
    pltpu.sync_copy(he_hbm.at[pl.ds(base * 128, _ROWS * 128)],
                    val_v.at[pl.ds(0, _ROWS * 128)])

    z16f = jnp.zeros((16,), jnp.float32)
    z16i = jnp.zeros((16,), jnp.int32)
    for i in range(8):
        idx_v[_ROWS, pl.ds(i * 16, 16)] = z16i
        val_v[pl.ds(_ROWS * 128 + i * 16, 16)] = z16f

    @pl.when(cnt == 390)
    def _():
        for i in range(8):
            val_v[pl.ds((_ROWS - 1) * 128 + i * 16, 16)] = z16f

    def zero_body(i, carry):
        zbuf[pl.ds(i * 16, 16)] = z16f
        return carry

    lax.fori_loop(0, _CHUNK // 16, zero_body, 0)
    pltpu.sync_copy(zbuf, acc.at[pl.ds(s * _CHUNK, _CHUNK)])
    plsc.subcore_barrier()

    def scat_body(j, carry):
        pltpu.sync_copy(val_v.at[pl.ds(j * 128, 128)],
                        acc.at[idx_v.at[j]], add=True)
        return carry

    lax.fori_loop(0, _ROWS + 1, scat_body, 0)
    plsc.subcore_barrier()

    pltpu.sync_copy(acc.at[pl.ds(s * _CHUNK, _CHUNK)], zbuf)
    pltpu.sync_copy(zbuf, out_hbm.at[c].at[pl.ds(s * _CHUNK, _CHUNK)])


@functools.cache
def _make_sc_segment():
    return pl.kernel(
        _sc_segment_body,
        out_type=jax.ShapeDtypeStruct((2, _NPAD), jnp.float32),
        mesh=plsc.VectorSubcoreMesh(core_axis_name="c", subcore_axis_name="s"),
        compiler_params=pltpu.CompilerParams(use_tc_tiling_on_sc=False),
        scratch_types=[
            pltpu.VMEM((_ROWS + 1, 128), jnp.int32),
            pltpu.VMEM(((_ROWS + 1) * 128,), jnp.float32),
            pltpu.VMEM((_CHUNK,), jnp.float32),
            pltpu.VMEM_SHARED((_NPAD,), jnp.float32),
        ],
    )


_BN = 2048


def _mlp_body(pt_ref, hn_ref, w1a_ref, w1b_ref, b1_ref, w2_ref, b2_ref,
              w3t_ref, b3t_ref, out_ref):
    x = hn_ref[...]
    h = jnp.dot(x, w1a_ref[...], preferred_element_type=jnp.float32)
    agg = pt_ref[...].reshape(_BN, 1)
    h = h + agg * w1b_ref[...]
    h = jnp.maximum(h + b1_ref[...], 0.0)
    h = jnp.dot(h, w2_ref[...], preferred_element_type=jnp.float32)
    h = jnp.maximum(h + b2_ref[...], 0.0)
    out_ref[...] = lax.dot_general(
        w3t_ref[...], h, (((1,), (1,)), ((), ())),
        preferred_element_type=jnp.float32,
    ) + b3t_ref[...]


def _mlp_tc(pt, hn, w1a, w1b, b1, w2, b2, w3t, b3t):
    n_out = w3t.shape[0]
    grid = (pl.cdiv(_N, _BN),)
    return pl.pallas_call(
        _mlp_body,
        grid=grid,
        in_specs=[
            pl.BlockSpec((_BN,), lambda i: (i,)),
            pl.BlockSpec((_BN, _F), lambda i: (i, 0)),
            pl.BlockSpec((_F, _F), lambda i: (0, 0)),
            pl.BlockSpec((1, _F), lambda i: (0, 0)),
            pl.BlockSpec((1, _F), lambda i: (0, 0)),
            pl.BlockSpec((_F, _F), lambda i: (0, 0)),
            pl.BlockSpec((1, _F), lambda i: (0, 0)),
            pl.BlockSpec((n_out, _F), lambda i: (0, 0)),
            pl.BlockSpec((n_out, 1), lambda i: (0, 0)),
        ],
        out_specs=pl.BlockSpec((n_out, _BN), lambda i: (0, i)),
        out_shape=jax.ShapeDtypeStruct((n_out, _N), jnp.float32),
    )(pt, hn, w1a, w1b, b1, w2, b2, w3t, b3t)


def kernel(hn, he, edge_index, W1, b1, W2, b2, W3, b3):
    dst = edge_index[1].astype(jnp.int32).reshape(_R, 128)
    he1 = he.reshape(_E)

    partials = _make_sc_segment()(dst, he1)
    pt = partials[0, :_N] + partials[1, :_N]

    w1a = W1[:_F]
    w1b = W1[_F:_F + 1]
    out3 = _mlp_tc(
        pt, hn, w1a, w1b, b1.reshape(1, _F), W2, b2.reshape(1, _F),
        W3.T, b3.reshape(-1, 1),
    )
    return out3.T


# speedup vs baseline: 11.3179x; 1.3688x over previous
import functools

import jax
import jax.numpy as jnp
from jax import lax
from jax.experimental import pallas as pl
from jax.experimental.pallas import tpu as pltpu
from jax.experimental.pallas import tpu_sc as plsc

_N = 100000
_E = 1600000
_F = 128
_R = _E // 128
_NW = 32
_ROWS = 391
_NPAD = 100096
_CHUNK = _NPAD // 16


_LROWS = 392
_HROWS = _LROWS // 2


def _zero_val_row(val_v, r):
    z16f = jnp.zeros((16,), jnp.float32)
    for i in range(8):
        val_v[pl.ds(r * 128 + i * 16, 16)] = z16f


def _sc_segment_body(ei_hbm, he_hbm, out_hbm, ei_v, val_v, zbuf, acc):
    c = lax.axis_index("c")
    s = lax.axis_index("s")
    w = s * 2 + c

    base = (w * _R) // _NW
    cnt = ((w + 1) * _R) // _NW - base
    load_base = jnp.minimum(base, _R - _LROWS)
    off = base - load_base

    pltpu.sync_copy(he_hbm.at[pl.ds(load_base * 128, _LROWS * 128)], val_v)

    @pl.when(off == 1)
    def _():
        _zero_val_row(val_v, 0)

    @pl.when(off + cnt <= _LROWS - 1)
    def _():
        _zero_val_row(val_v, _LROWS - 1)

    @pl.when(off + cnt <= _LROWS - 2)
    def _():
        _zero_val_row(val_v, _LROWS - 2)

    z16f = jnp.zeros((16,), jnp.float32)

    def zero_body(i, carry):
        zbuf[pl.ds(i * 16, 16)] = z16f
        return carry

    lax.fori_loop(0, _CHUNK // 16, zero_body, 0)
    pltpu.sync_copy(zbuf, acc.at[pl.ds(s * _CHUNK, _CHUNK)])
    plsc.subcore_barrier()

    for h in range(2):
        pltpu.sync_copy(ei_hbm.at[pl.ds(load_base + h * _HROWS, _HROWS)],
                        ei_v)

        def scat_body(j, carry, _h=h):
            pltpu.sync_copy(val_v.at[pl.ds((_h * _HROWS + j) * 128, 128)],
                            acc.at[ei_v.at[j, 1]], add=True)
            return carry

        lax.fori_loop(0, _HROWS, scat_body, 0)
    plsc.subcore_barrier()

    pltpu.sync_copy(acc.at[pl.ds(s * _CHUNK, _CHUNK)], zbuf)
    pltpu.sync_copy(zbuf, out_hbm.at[c].at[pl.ds(s * _CHUNK, _CHUNK)])


@functools.cache
def _make_sc_segment():
    return pl.kernel(
        _sc_segment_body,
        out_type=jax.ShapeDtypeStruct((2, _NPAD), jnp.float32),
        mesh=plsc.VectorSubcoreMesh(core_axis_name="c", subcore_axis_name="s"),
        compiler_params=pltpu.CompilerParams(use_tc_tiling_on_sc=False),
        scratch_types=[
            pltpu.VMEM((_HROWS, 2, 128), jnp.int32),
            pltpu.VMEM((_LROWS * 128,), jnp.float32),
            pltpu.VMEM((_CHUNK,), jnp.float32),
            pltpu.VMEM_SHARED((_NPAD,), jnp.float32),
        ],
    )


_BN = 2048


def _mlp_body(pt_ref, hn_ref, w1a_ref, w1b_ref, b1_ref, w2_ref, b2_ref,
              w3t_ref, b3t_ref, out_ref):
    x = hn_ref[...]
    h = jnp.dot(x, w1a_ref[...], preferred_element_type=jnp.float32)
    agg = pt_ref[...].reshape(_BN, 1)
    h = h + agg * w1b_ref[...]
    h = jnp.maximum(h + b1_ref[...], 0.0)
    h = jnp.dot(h, w2_ref[...], preferred_element_type=jnp.float32)
    h = jnp.maximum(h + b2_ref[...], 0.0)
    out_ref[...] = lax.dot_general(
        w3t_ref[...], h, (((1,), (1,)), ((), ())),
        preferred_element_type=jnp.float32,
    ) + b3t_ref[...]


def _mlp_tc(pt, hn, w1a, w1b, b1, w2, b2, w3t, b3t):
    n_out = w3t.shape[0]
    grid = (pl.cdiv(_N, _BN),)
    return pl.pallas_call(
        _mlp_body,
        grid=grid,
        in_specs=[
            pl.BlockSpec((_BN,), lambda i: (i,)),
            pl.BlockSpec((_BN, _F), lambda i: (i, 0)),
            pl.BlockSpec((_F, _F), lambda i: (0, 0)),
            pl.BlockSpec((1, _F), lambda i: (0, 0)),
            pl.BlockSpec((1, _F), lambda i: (0, 0)),
            pl.BlockSpec((_F, _F), lambda i: (0, 0)),
            pl.BlockSpec((1, _F), lambda i: (0, 0)),
            pl.BlockSpec((n_out, _F), lambda i: (0, 0)),
            pl.BlockSpec((n_out, 1), lambda i: (0, 0)),
        ],
        out_specs=pl.BlockSpec((n_out, _BN), lambda i: (0, i)),
        out_shape=jax.ShapeDtypeStruct((n_out, _N), jnp.float32),
    )(pt, hn, w1a, w1b, b1, w2, b2, w3t, b3t)


def kernel(hn, he, edge_index, W1, b1, W2, b2, W3, b3):
    ei3 = edge_index.astype(jnp.int32).reshape(2, _R, 128).transpose(1, 0, 2)
    he1 = he.reshape(_E)
    partials = _make_sc_segment()(ei3, he1)
    pt = partials[0, :_N] + partials[1, :_N]

    w1a = W1[:_F]
    w1b = W1[_F:_F + 1]
    out3 = _mlp_tc(
        pt, hn, w1a, w1b, b1.reshape(1, _F), W2, b2.reshape(1, _F),
        W3.T, b3.reshape(-1, 1),
    )
    return out3.T
